# Initial kernel scaffold; baseline (speedup 1.0000x reference)
import functools

import jax
import jax.numpy as jnp
from jax import lax
from jax.experimental import pallas as pl
from jax.experimental.pallas import tpu as pltpu
from jax.experimental.pallas import tpu_sc as plsc

N = 10000
E = 320000
D = 128
R = 9
H = 128

NC = 2
NS = 16
NW = NC * NS

GCH = 40
EPW = E // NW
GSTEPS = EPW // GCH

_SC_MESH = plsc.VectorSubcoreMesh(core_axis_name="c", subcore_axis_name="s")


@functools.partial(
    pl.kernel,
    out_type=jax.ShapeDtypeStruct((2 * E, D), jnp.float32),
    mesh=_SC_MESH,
    scratch_types=[
        pltpu.VMEM((2 * GCH,), jnp.int32),
        pltpu.VMEM((2 * GCH, D), jnp.float32),
        pltpu.SemaphoreType.DMA,
    ],
)
def _sc_gather(idx_hbm, node_hbm, out_hbm, idx_v, rows_v, sem):
    cid = lax.axis_index("c")
    sid = lax.axis_index("s")
    wid = sid * NC + cid
    base = wid * EPW

    def step(k, carry):
        off = 2 * (base + k * GCH)
        pltpu.sync_copy(idx_hbm.at[pl.ds(off, 2 * GCH)], idx_v)
        pltpu.async_copy(node_hbm.at[idx_v], rows_v, sem).wait()
        pltpu.sync_copy(rows_v, out_hbm.at[pl.ds(off, 2 * GCH)])
        return carry

    lax.fori_loop(0, GSTEPS, step, 0)


BE = 512
GRID = E // BE


def _silu(x):
    return x * jax.nn.sigmoid(x)


def _mlp_body(vivj_ref, ef_ref, rbf_ref,
              eW1, eb1, eW2, eb2, eG1, eg1, eG2, eg2,
              nW1, nb1, nW2, nb2, nG1, ng1, nG2, ng2,
              We, Wv, enew_ref, mess_ref):
    ef = ef_ref[...]
    x = jnp.concatenate([vivj_ref[...], ef], axis=1)
    rbf = rbf_ref[...]

    h = _silu(jnp.dot(x, eW1[...], preferred_element_type=jnp.float32) + eb1[...])
    h = _silu(jnp.dot(h, eW2[...], preferred_element_type=jnp.float32) + eb2[...])
    g = _silu(jnp.dot(x, eG1[...], preferred_element_type=jnp.float32) + eg1[...])
    g = jax.nn.sigmoid(jnp.dot(g, eG2[...], preferred_element_type=jnp.float32) + eg2[...])
    mij = h * g * jnp.dot(rbf, We[...], preferred_element_type=jnp.float32)
    e_new = ef + mij
    enew_ref[...] = e_new

    xv = jnp.concatenate([vivj_ref[...], e_new], axis=1)
    p = _silu(jnp.dot(xv, nW1[...], preferred_element_type=jnp.float32) + nb1[...])
    p = _silu(jnp.dot(p, nW2[...], preferred_element_type=jnp.float32) + nb2[...])
    q = _silu(jnp.dot(xv, nG1[...], preferred_element_type=jnp.float32) + ng1[...])
    q = jax.nn.sigmoid(jnp.dot(q, nG2[...], preferred_element_type=jnp.float32) + ng2[...])
    mess_ref[...] = p * q * jnp.dot(rbf, Wv[...], preferred_element_type=jnp.float32)


def _edge_block(i):
    return (i, 0)


def _fixed(i):
    return (0, 0)


def _tc_mlp(vivj, edge_feat, rbf, weights):
    wspecs = [pl.BlockSpec(w.shape, _fixed) for w in weights]
    return pl.pallas_call(
        _mlp_body,
        grid=(GRID,),
        in_specs=[
            pl.BlockSpec((BE, 2 * D), _edge_block),
            pl.BlockSpec((BE, D), _edge_block),
            pl.BlockSpec((BE, R), _edge_block),
            *wspecs,
        ],
        out_specs=[
            pl.BlockSpec((BE, D), _edge_block),
            pl.BlockSpec((BE, D), _edge_block),
        ],
        out_shape=[
            jax.ShapeDtypeStruct((E, D), jnp.float32),
            jax.ShapeDtypeStruct((E, D), jnp.float32),
        ],
        compiler_params=pltpu.CompilerParams(
            dimension_semantics=("arbitrary",),
        ),
    )(vivj, edge_feat, rbf, *weights)


SCH = 80
SSTEPS = EPW // SCH
NPS = N // NS


@functools.partial(
    pl.kernel,
    out_type=jax.ShapeDtypeStruct((2 * N, D), jnp.float32),
    mesh=_SC_MESH,
    scratch_types=[
        pltpu.VMEM((SCH,), jnp.int32),
        pltpu.VMEM((SCH, D), jnp.float32),
        pltpu.VMEM_SHARED((N, D), jnp.float32),
        pltpu.SemaphoreType.DMA,
    ],
)
def _sc_scatter(mess_hbm, dst_hbm, nfh_hbm, out_hbm, idx_v, rows_v, acc_sh, sem):
    cid = lax.axis_index("c")
    sid = lax.axis_index("s")
    pltpu.sync_copy(nfh_hbm.at[pl.ds(sid * NPS, NPS)],
                    acc_sh.at[pl.ds(sid * NPS, NPS)])
    plsc.subcore_barrier()

    base = cid * (E // NC) + sid * EPW

    def step(k, carry):
        off = base + k * SCH
        pltpu.sync_copy(dst_hbm.at[pl.ds(off, SCH)], idx_v)
        pltpu.sync_copy(mess_hbm.at[pl.ds(off, SCH)], rows_v)
        pltpu.sync_copy(rows_v, acc_sh.at[idx_v], add=True)
        return carry

    lax.fori_loop(0, SSTEPS, step, 0)
    plsc.subcore_barrier()
    pltpu.sync_copy(acc_sh.at[pl.ds(sid * NPS, NPS)],
                    out_hbm.at[pl.ds(cid * N + sid * NPS, NPS)])


def kernel(node_feat, edge_feat, rbf, edge_index,
           eW1, eb1, eW2, eb2, eG1, eg1, eG2, eg2,
           nW1, nb1, nW2, nb2, nG1, ng1, nG2, ng2,
           We, Wv):
    idx2 = edge_index.astype(jnp.int32).T.reshape(2 * E)
    dst = edge_index[1].astype(jnp.int32)

    vivj = _sc_gather(idx2, node_feat).reshape(E, 2 * D)

    weights = (eW1, eb1.reshape(1, H), eW2, eb2.reshape(1, H),
               eG1, eg1.reshape(1, H), eG2, eg2.reshape(1, H),
               nW1, nb1.reshape(1, H), nW2, nb2.reshape(1, H),
               nG1, ng1.reshape(1, H), nG2, ng2.reshape(1, H),
               We, Wv)
    e_new, mess = _tc_mlp(vivj, edge_feat, rbf, weights)

    parts = _sc_scatter(mess, dst, node_feat * 0.5)
    v_new = parts[:N] + parts[N:]
    return (e_new, v_new)


# speedup vs baseline: 1.6882x; 1.6882x over previous
import functools

import jax
import jax.numpy as jnp
from jax import lax
from jax.experimental import pallas as pl
from jax.experimental.pallas import tpu as pltpu
from jax.experimental.pallas import tpu_sc as plsc

N = 10000
E = 320000
D = 128
R = 9
H = 128

NC = 2
NS = 16
NW = NC * NS

GCH = 40
EPW = E // NW
GSTEPS = EPW // GCH

_SC_MESH = plsc.VectorSubcoreMesh(core_axis_name="c", subcore_axis_name="s")


@functools.partial(
    pl.kernel,
    out_type=jax.ShapeDtypeStruct((2 * E, D), jnp.float32),
    mesh=_SC_MESH,
    scratch_types=[
        pltpu.VMEM((2 * GCH,), jnp.int32),
        pltpu.VMEM((2 * GCH, D), jnp.float32),
        pltpu.SemaphoreType.DMA,
    ],
)
def _sc_gather(idx_hbm, node_hbm, out_hbm, idx_v, rows_v, sem):
    cid = lax.axis_index("c")
    sid = lax.axis_index("s")
    wid = sid * NC + cid
    base = wid * EPW

    def step(k, carry):
        off = 2 * (base + k * GCH)
        pltpu.sync_copy(idx_hbm.at[pl.ds(off, 2 * GCH)], idx_v)
        pltpu.async_copy(node_hbm.at[idx_v], rows_v, sem).wait()
        pltpu.sync_copy(rows_v, out_hbm.at[pl.ds(off, 2 * GCH)])
        return carry

    lax.fori_loop(0, GSTEPS, step, 0)


BE = 512
GRID = E // BE


def _silu(x):
    return x * jax.nn.sigmoid(x)


def _mlp_body(vivj_ref, ef_ref, rbf_ref,
              eW1, eb1, eW2, eb2, eG1, eg1, eG2, eg2,
              nW1, nb1, nW2, nb2, nG1, ng1, nG2, ng2,
              We, Wv, enew_ref, mess_ref):
    ef = ef_ref[...]
    x = jnp.concatenate([vivj_ref[...], ef], axis=1)
    rbf = rbf_ref[...]

    h = _silu(jnp.dot(x, eW1[...], preferred_element_type=jnp.float32) + eb1[...])
    h = _silu(jnp.dot(h, eW2[...], preferred_element_type=jnp.float32) + eb2[...])
    g = _silu(jnp.dot(x, eG1[...], preferred_element_type=jnp.float32) + eg1[...])
    g = jax.nn.sigmoid(jnp.dot(g, eG2[...], preferred_element_type=jnp.float32) + eg2[...])
    mij = h * g * jnp.dot(rbf, We[...], preferred_element_type=jnp.float32)
    e_new = ef + mij
    enew_ref[...] = e_new

    xv = jnp.concatenate([vivj_ref[...], e_new], axis=1)
    p = _silu(jnp.dot(xv, nW1[...], preferred_element_type=jnp.float32) + nb1[...])
    p = _silu(jnp.dot(p, nW2[...], preferred_element_type=jnp.float32) + nb2[...])
    q = _silu(jnp.dot(xv, nG1[...], preferred_element_type=jnp.float32) + ng1[...])
    q = jax.nn.sigmoid(jnp.dot(q, nG2[...], preferred_element_type=jnp.float32) + ng2[...])
    mess_ref[...] = p * q * jnp.dot(rbf, Wv[...], preferred_element_type=jnp.float32)


def _edge_block(i):
    return (i, 0)


def _fixed(i):
    return (0, 0)


def _tc_mlp(vivj, edge_feat, rbf, weights):
    wspecs = [pl.BlockSpec(w.shape, _fixed) for w in weights]
    return pl.pallas_call(
        _mlp_body,
        grid=(GRID,),
        in_specs=[
            pl.BlockSpec((BE, 2 * D), _edge_block),
            pl.BlockSpec((BE, D), _edge_block),
            pl.BlockSpec((BE, R), _edge_block),
            *wspecs,
        ],
        out_specs=[
            pl.BlockSpec((BE, D), _edge_block),
            pl.BlockSpec((BE, D), _edge_block),
        ],
        out_shape=[
            jax.ShapeDtypeStruct((E, D), jnp.float32),
            jax.ShapeDtypeStruct((E, D), jnp.float32),
        ],
        compiler_params=pltpu.CompilerParams(
            dimension_semantics=("arbitrary",),
        ),
    )(vivj, edge_feat, rbf, *weights)


SCH = 80
SSTEPS = EPW // SCH
NPS = 632
N_PAD = NPS * NS


@functools.partial(
    pl.kernel,
    out_type=jax.ShapeDtypeStruct((2 * N_PAD, D), jnp.float32),
    mesh=_SC_MESH,
    scratch_types=[
        pltpu.VMEM((SCH,), jnp.int32),
        pltpu.VMEM((SCH, D), jnp.float32),
        pltpu.VMEM_SHARED((N_PAD, D), jnp.float32),
        pltpu.SemaphoreType.DMA,
    ],
)
def _sc_scatter(mess_hbm, dst_hbm, nfh_hbm, out_hbm, idx_v, rows_v, acc_sh, sem):
    cid = lax.axis_index("c")
    sid = lax.axis_index("s")
    pltpu.sync_copy(nfh_hbm.at[pl.ds(sid * NPS, NPS)],
                    acc_sh.at[pl.ds(sid * NPS, NPS)])
    plsc.subcore_barrier()

    base = cid * (E // NC) + sid * EPW

    def step(k, carry):
        off = base + k * SCH
        pltpu.sync_copy(dst_hbm.at[pl.ds(off, SCH)], idx_v)
        pltpu.sync_copy(mess_hbm.at[pl.ds(off, SCH)], rows_v)
        pltpu.sync_copy(rows_v, acc_sh.at[idx_v], add=True)
        return carry

    lax.fori_loop(0, SSTEPS, step, 0)
    plsc.subcore_barrier()
    pltpu.sync_copy(acc_sh.at[pl.ds(sid * NPS, NPS)],
                    out_hbm.at[pl.ds(cid * N_PAD + sid * NPS, NPS)])


def kernel(node_feat, edge_feat, rbf, edge_index,
           eW1, eb1, eW2, eb2, eG1, eg1, eG2, eg2,
           nW1, nb1, nW2, nb2, nG1, ng1, nG2, ng2,
           We, Wv):
    idx2 = edge_index.astype(jnp.int32).T.reshape(2 * E)
    dst = edge_index[1].astype(jnp.int32)

    vivj = _sc_gather(idx2, node_feat).reshape(E, 2 * D)

    weights = (eW1, eb1.reshape(1, H), eW2, eb2.reshape(1, H),
               eG1, eg1.reshape(1, H), eG2, eg2.reshape(1, H),
               nW1, nb1.reshape(1, H), nW2, nb2.reshape(1, H),
               nG1, ng1.reshape(1, H), nG2, ng2.reshape(1, H),
               We, Wv)
    e_new, mess = _tc_mlp(vivj, edge_feat, rbf, weights)

    nfh = jnp.zeros((N_PAD, D), jnp.float32).at[:N].set(node_feat * 0.5)
    parts = _sc_scatter(mess, dst, nfh)
    v_new = parts[:N] + parts[N_PAD:N_PAD + N]
    return (e_new, v_new)
